# baseline (device time: 49191 ns/iter reference)
import jax
import jax.numpy as jnp
from jax import lax
from jax.experimental import pallas as pl
from jax.experimental.pallas import tpu as pltpu

N_DEV = 4
SUB = 256
LOG2E = 1.4426950408889634


def kernel(q, k, v):
    s_loc, d = q.shape
    half = s_loc // 2
    n_sub = half // SUB
    scale = LOG2E / (d ** 0.5)

    def body(q_ref, k_ref, v_ref, out_ref,
             kcw, vcw, kccw, vccw,
             kcw_s, kcw_r, vcw_s, vcw_r,
             kccw_s, kccw_r, vccw_s, vccw_r):
        my = lax.axis_index("i")
        left = (my - 1) % N_DEV
        right = (my + 1) % N_DEV

        barrier_sem = pltpu.get_barrier_semaphore()
        for nbr in [left, right]:
            pl.semaphore_signal(
                barrier_sem, inc=1,
                device_id=(nbr,), device_id_type=pl.DeviceIdType.MESH,
            )
        pl.semaphore_wait(barrier_sem, 2)

        n_qt = s_loc // SUB
        q_t = [
            q_ref[pl.ds(t * SUB, SUB), :].astype(jnp.bfloat16)
            for t in range(n_qt)
        ]
        kcw[0] = k_ref[pl.ds(0, half), :].astype(jnp.bfloat16)
        vcw[0] = v_ref[pl.ds(0, half), :].astype(jnp.bfloat16)
        kccw[0] = k_ref[pl.ds(half, half), :].astype(jnp.bfloat16)
        vccw[0] = v_ref[pl.ds(half, half), :].astype(jnp.bfloat16)

        started = []

        def forward(buf, send_sems, recv_sems, s, c, dst):
            rdma = pltpu.make_async_remote_copy(
                src_ref=buf.at[s, pl.ds(c * SUB, SUB)],
                dst_ref=buf.at[s + 1, pl.ds(c * SUB, SUB)],
                send_sem=send_sems.at[s * n_sub + c],
                recv_sem=recv_sems.at[(s + 1) * n_sub + c],
                device_id=(dst,), device_id_type=pl.DeviceIdType.MESH,
            )
            rdma.start()
            started.append(rdma)

        def wait_chunk(buf, send_sems, recv_sems, s, c):
            rdma = pltpu.make_async_remote_copy(
                src_ref=buf.at[s, pl.ds(c * SUB, SUB)],
                dst_ref=buf.at[s, pl.ds(c * SUB, SUB)],
                send_sem=send_sems.at[s * n_sub + c],
                recv_sem=recv_sems.at[s * n_sub + c],
                device_id=(left,), device_id_type=pl.DeviceIdType.MESH,
            )
            rdma.wait_recv()

        for c in range(n_sub):
            forward(kcw, kcw_s, kcw_r, 0, c, right)
            forward(vcw, vcw_s, vcw_r, 0, c, right)
            forward(kccw, kccw_s, kccw_r, 0, c, left)
            forward(vccw, vccw_s, vccw_r, 0, c, left)

        l_t = [jnp.zeros((SUB, 1), dtype=jnp.float32) for _ in range(n_qt)]
        acc_t = [jnp.zeros((SUB, d), dtype=jnp.float32) for _ in range(n_qt)]

        def update(k_sub, v_sub):
            for t in range(n_qt):
                s = lax.dot_general(
                    q_t[t], k_sub, (((1,), (1,)), ((), ())),
                    preferred_element_type=jnp.float32,
                ) * scale
                p = jnp.exp2(s)
                l_t[t] = l_t[t] + jnp.sum(p, axis=1, keepdims=True)
                pv = lax.dot_general(
                    p.astype(jnp.bfloat16), v_sub, (((1,), (0,)), ((), ())),
                    preferred_element_type=jnp.float32,
                )
                acc_t[t] = acc_t[t] + pv

        def sub(buf, s, c):
            return buf[s, pl.ds(c * SUB, SUB), :]

        for c in range(n_sub):
            update(sub(kcw, 0, c), sub(vcw, 0, c))
            update(sub(kccw, 0, c), sub(vccw, 0, c))

        for s in range(1, N_DEV):
            for c in range(n_sub):
                wait_chunk(kcw, kcw_s, kcw_r, s, c)
                wait_chunk(vcw, vcw_s, vcw_r, s, c)
                if s < N_DEV - 1:
                    forward(kcw, kcw_s, kcw_r, s, c, right)
                    forward(vcw, vcw_s, vcw_r, s, c, right)
                update(sub(kcw, s, c), sub(vcw, s, c))

                wait_chunk(kccw, kccw_s, kccw_r, s, c)
                wait_chunk(vccw, vccw_s, vccw_r, s, c)
                if s < N_DEV - 1:
                    forward(kccw, kccw_s, kccw_r, s, c, left)
                    forward(vccw, vccw_s, vccw_r, s, c, left)
                update(sub(kccw, s, c), sub(vccw, s, c))

        for t in range(n_qt):
            out_ref[pl.ds(t * SUB, SUB), :] = acc_t[t] / l_t[t]

        for rdma in started:
            rdma.wait_send()

    dma_sc = pltpu.SemaphoreType.DMA((N_DEV * n_sub,))
    return pl.pallas_call(
        body,
        out_shape=jax.ShapeDtypeStruct((s_loc, d), jnp.float32),
        in_specs=[
            pl.BlockSpec(memory_space=pltpu.VMEM),
            pl.BlockSpec(memory_space=pltpu.VMEM),
            pl.BlockSpec(memory_space=pltpu.VMEM),
        ],
        out_specs=pl.BlockSpec(memory_space=pltpu.VMEM),
        scratch_shapes=[
            pltpu.VMEM((N_DEV, half, d), jnp.bfloat16),
            pltpu.VMEM((N_DEV, half, d), jnp.bfloat16),
            pltpu.VMEM((N_DEV, half, d), jnp.bfloat16),
            pltpu.VMEM((N_DEV, half, d), jnp.bfloat16),
            dma_sc, dma_sc, dma_sc, dma_sc,
            dma_sc, dma_sc, dma_sc, dma_sc,
        ],
        compiler_params=pltpu.CompilerParams(collective_id=0),
    )(q, k, v)


# device time: 48905 ns/iter; 1.0058x vs baseline; 1.0058x over previous
import jax
import jax.numpy as jnp
from jax import lax
from jax.experimental import pallas as pl
from jax.experimental.pallas import tpu as pltpu

N_DEV = 4
SUB = 512
QT = 1024
LOG2E = 1.4426950408889634


def kernel(q, k, v):
    s_loc, d = q.shape
    half = s_loc // 2
    n_sub = half // SUB
    scale = LOG2E / (d ** 0.5)

    def body(q_ref, k_ref, v_ref, out_ref,
             kcw, vcw, kccw, vccw,
             kcw_s, kcw_r, vcw_s, vcw_r,
             kccw_s, kccw_r, vccw_s, vccw_r):
        my = lax.axis_index("i")
        left = (my - 1) % N_DEV
        right = (my + 1) % N_DEV

        barrier_sem = pltpu.get_barrier_semaphore()
        for nbr in [left, right]:
            pl.semaphore_signal(
                barrier_sem, inc=1,
                device_id=(nbr,), device_id_type=pl.DeviceIdType.MESH,
            )
        pl.semaphore_wait(barrier_sem, 2)

        n_qt = s_loc // QT
        q_t = [
            q_ref[pl.ds(t * QT, QT), :].astype(jnp.bfloat16)
            for t in range(n_qt)
        ]
        kcw[0] = k_ref[pl.ds(0, half), :].astype(jnp.bfloat16)
        vcw[0] = v_ref[pl.ds(0, half), :].astype(jnp.bfloat16)
        kccw[0] = k_ref[pl.ds(half, half), :].astype(jnp.bfloat16)
        vccw[0] = v_ref[pl.ds(half, half), :].astype(jnp.bfloat16)

        started = []

        def forward(buf, send_sems, recv_sems, s, c, dst):
            rdma = pltpu.make_async_remote_copy(
                src_ref=buf.at[s, pl.ds(c * SUB, SUB)],
                dst_ref=buf.at[s + 1, pl.ds(c * SUB, SUB)],
                send_sem=send_sems.at[s * n_sub + c],
                recv_sem=recv_sems.at[(s + 1) * n_sub + c],
                device_id=(dst,), device_id_type=pl.DeviceIdType.MESH,
            )
            rdma.start()
            started.append(rdma)

        def wait_chunk(buf, send_sems, recv_sems, s, c):
            rdma = pltpu.make_async_remote_copy(
                src_ref=buf.at[s, pl.ds(c * SUB, SUB)],
                dst_ref=buf.at[s, pl.ds(c * SUB, SUB)],
                send_sem=send_sems.at[s * n_sub + c],
                recv_sem=recv_sems.at[s * n_sub + c],
                device_id=(left,), device_id_type=pl.DeviceIdType.MESH,
            )
            rdma.wait_recv()

        for c in range(n_sub):
            forward(kcw, kcw_s, kcw_r, 0, c, right)
            forward(vcw, vcw_s, vcw_r, 0, c, right)
            forward(kccw, kccw_s, kccw_r, 0, c, left)
            forward(vccw, vccw_s, vccw_r, 0, c, left)

        l_t = [jnp.zeros((QT, 1), dtype=jnp.float32) for _ in range(n_qt)]
        acc_t = [jnp.zeros((QT, d), dtype=jnp.float32) for _ in range(n_qt)]

        def update(k_sub, v_sub):
            for t in range(n_qt):
                s = lax.dot_general(
                    q_t[t], k_sub, (((1,), (1,)), ((), ())),
                    preferred_element_type=jnp.float32,
                ) * scale
                p = jnp.exp2(s)
                l_t[t] = l_t[t] + jnp.sum(p, axis=1, keepdims=True)
                pv = lax.dot_general(
                    p.astype(jnp.bfloat16), v_sub, (((1,), (0,)), ((), ())),
                    preferred_element_type=jnp.float32,
                )
                acc_t[t] = acc_t[t] + pv

        def sub(buf, s, c):
            return buf[s, pl.ds(c * SUB, SUB), :]

        for c in range(n_sub):
            update(sub(kcw, 0, c), sub(vcw, 0, c))
            update(sub(kccw, 0, c), sub(vccw, 0, c))

        for s in range(1, N_DEV):
            for c in range(n_sub):
                wait_chunk(kcw, kcw_s, kcw_r, s, c)
                wait_chunk(vcw, vcw_s, vcw_r, s, c)
                if s < N_DEV - 1:
                    forward(kcw, kcw_s, kcw_r, s, c, right)
                    forward(vcw, vcw_s, vcw_r, s, c, right)
                update(sub(kcw, s, c), sub(vcw, s, c))

                wait_chunk(kccw, kccw_s, kccw_r, s, c)
                wait_chunk(vccw, vccw_s, vccw_r, s, c)
                if s < N_DEV - 1:
                    forward(kccw, kccw_s, kccw_r, s, c, left)
                    forward(vccw, vccw_s, vccw_r, s, c, left)
                update(sub(kccw, s, c), sub(vccw, s, c))

        for t in range(n_qt):
            out_ref[pl.ds(t * QT, QT), :] = acc_t[t] / l_t[t]

        for rdma in started:
            rdma.wait_send()

    dma_sc = pltpu.SemaphoreType.DMA((N_DEV * n_sub,))
    return pl.pallas_call(
        body,
        out_shape=jax.ShapeDtypeStruct((s_loc, d), jnp.float32),
        in_specs=[
            pl.BlockSpec(memory_space=pltpu.VMEM),
            pl.BlockSpec(memory_space=pltpu.VMEM),
            pl.BlockSpec(memory_space=pltpu.VMEM),
        ],
        out_specs=pl.BlockSpec(memory_space=pltpu.VMEM),
        scratch_shapes=[
            pltpu.VMEM((N_DEV, half, d), jnp.bfloat16),
            pltpu.VMEM((N_DEV, half, d), jnp.bfloat16),
            pltpu.VMEM((N_DEV, half, d), jnp.bfloat16),
            pltpu.VMEM((N_DEV, half, d), jnp.bfloat16),
            dma_sc, dma_sc, dma_sc, dma_sc,
            dma_sc, dma_sc, dma_sc, dma_sc,
        ],
        compiler_params=pltpu.CompilerParams(collective_id=0),
    )(q, k, v)


# device time: 48503 ns/iter; 1.0142x vs baseline; 1.0083x over previous
import jax
import jax.numpy as jnp
from jax import lax
from jax.experimental import pallas as pl
from jax.experimental.pallas import tpu as pltpu

N_DEV = 4
SUB = 512
QT = 1024
LOG2E = 1.4426950408889634


def kernel(q, k, v):
    s_loc, d = q.shape
    half = s_loc // 2
    n_sub = half // SUB
    scale = LOG2E / (d ** 0.5)

    def body(q_ref, k_ref, v_ref, out_ref,
             kcw, vcw, kccw, vccw,
             kcw_s, kcw_r, vcw_s, vcw_r,
             kccw_s, kccw_r, vccw_s, vccw_r):
        my = lax.axis_index("i")
        left = (my - 1) % N_DEV
        right = (my + 1) % N_DEV

        barrier_sem = pltpu.get_barrier_semaphore()
        for nbr in [left, right]:
            pl.semaphore_signal(
                barrier_sem, inc=1,
                device_id=(nbr,), device_id_type=pl.DeviceIdType.MESH,
            )
        n_qt = s_loc // QT
        q_t = [
            q_ref[pl.ds(t * QT, QT), :].astype(jnp.bfloat16)
            for t in range(n_qt)
        ]
        kcw[0] = k_ref[pl.ds(0, half), :].astype(jnp.bfloat16)
        vcw[0] = v_ref[pl.ds(0, half), :].astype(jnp.bfloat16)
        kccw[0] = k_ref[pl.ds(half, half), :].astype(jnp.bfloat16)
        vccw[0] = v_ref[pl.ds(half, half), :].astype(jnp.bfloat16)
        pl.semaphore_wait(barrier_sem, 2)

        started = []

        def forward(buf, send_sems, recv_sems, s, c, dst):
            rdma = pltpu.make_async_remote_copy(
                src_ref=buf.at[s, pl.ds(c * SUB, SUB)],
                dst_ref=buf.at[s + 1, pl.ds(c * SUB, SUB)],
                send_sem=send_sems.at[s * n_sub + c],
                recv_sem=recv_sems.at[(s + 1) * n_sub + c],
                device_id=(dst,), device_id_type=pl.DeviceIdType.MESH,
            )
            rdma.start()
            started.append(rdma)

        def wait_chunk(buf, send_sems, recv_sems, s, c):
            rdma = pltpu.make_async_remote_copy(
                src_ref=buf.at[s, pl.ds(c * SUB, SUB)],
                dst_ref=buf.at[s, pl.ds(c * SUB, SUB)],
                send_sem=send_sems.at[s * n_sub + c],
                recv_sem=recv_sems.at[s * n_sub + c],
                device_id=(left,), device_id_type=pl.DeviceIdType.MESH,
            )
            rdma.wait_recv()

        for c in range(n_sub):
            forward(kcw, kcw_s, kcw_r, 0, c, right)
            forward(vcw, vcw_s, vcw_r, 0, c, right)
            forward(kccw, kccw_s, kccw_r, 0, c, left)
            forward(vccw, vccw_s, vccw_r, 0, c, left)

        l_t = [jnp.zeros((QT, 1), dtype=jnp.float32) for _ in range(n_qt)]
        acc_t = [jnp.zeros((QT, d), dtype=jnp.float32) for _ in range(n_qt)]

        def update(k_sub, v_sub):
            for t in range(n_qt):
                s = lax.dot_general(
                    q_t[t], k_sub, (((1,), (1,)), ((), ())),
                    preferred_element_type=jnp.float32,
                ) * scale
                p = jnp.exp2(s)
                l_t[t] = l_t[t] + jnp.sum(p, axis=1, keepdims=True)
                pv = lax.dot_general(
                    p.astype(jnp.bfloat16), v_sub, (((1,), (0,)), ((), ())),
                    preferred_element_type=jnp.float32,
                )
                acc_t[t] = acc_t[t] + pv

        def sub(buf, s, c):
            return buf[s, pl.ds(c * SUB, SUB), :]

        for c in range(n_sub):
            update(sub(kcw, 0, c), sub(vcw, 0, c))
            update(sub(kccw, 0, c), sub(vccw, 0, c))

        for s in range(1, N_DEV):
            for c in range(n_sub):
                wait_chunk(kcw, kcw_s, kcw_r, s, c)
                if s < N_DEV - 1:
                    forward(kcw, kcw_s, kcw_r, s, c, right)
                wait_chunk(vcw, vcw_s, vcw_r, s, c)
                if s < N_DEV - 1:
                    forward(vcw, vcw_s, vcw_r, s, c, right)
                update(sub(kcw, s, c), sub(vcw, s, c))

                wait_chunk(kccw, kccw_s, kccw_r, s, c)
                if s < N_DEV - 1:
                    forward(kccw, kccw_s, kccw_r, s, c, left)
                wait_chunk(vccw, vccw_s, vccw_r, s, c)
                if s < N_DEV - 1:
                    forward(vccw, vccw_s, vccw_r, s, c, left)
                update(sub(kccw, s, c), sub(vccw, s, c))

        for t in range(n_qt):
            out_ref[pl.ds(t * QT, QT), :] = acc_t[t] / l_t[t]

        for rdma in started:
            rdma.wait_send()

    dma_sc = pltpu.SemaphoreType.DMA((N_DEV * n_sub,))
    return pl.pallas_call(
        body,
        out_shape=jax.ShapeDtypeStruct((s_loc, d), jnp.float32),
        in_specs=[
            pl.BlockSpec(memory_space=pltpu.VMEM),
            pl.BlockSpec(memory_space=pltpu.VMEM),
            pl.BlockSpec(memory_space=pltpu.VMEM),
        ],
        out_specs=pl.BlockSpec(memory_space=pltpu.VMEM),
        scratch_shapes=[
            pltpu.VMEM((N_DEV, half, d), jnp.bfloat16),
            pltpu.VMEM((N_DEV, half, d), jnp.bfloat16),
            pltpu.VMEM((N_DEV, half, d), jnp.bfloat16),
            pltpu.VMEM((N_DEV, half, d), jnp.bfloat16),
            dma_sc, dma_sc, dma_sc, dma_sc,
            dma_sc, dma_sc, dma_sc, dma_sc,
        ],
        compiler_params=pltpu.CompilerParams(collective_id=0),
    )(q, k, v)
